# Initial kernel scaffold; baseline (speedup 1.0000x reference)
import jax
import jax.numpy as jnp
from jax.experimental import pallas as pl


def kernel(vectors, node_feats, node_specie, senders, receivers, W_up_s, W_up_v, W_up_t, mlp_w0, mlp_b0, mlp_w1, mlp_b1, mlp_w2, W_dn_s, W_dn_v, W_dn_t, W_sc_s, W_sc_v, W_sc_t):
    raise NotImplementedError("write your pallas kernel here")


# speedup vs baseline: 10.1890x; 10.1890x over previous
import functools

import jax
import jax.numpy as jnp
import numpy as np
from jax import lax
from jax.experimental import pallas as pl
from jax.experimental.pallas import tpu as pltpu
from jax.experimental.pallas import tpu_sc as plsc

FD = 8
N_NODES = 50000
E_EDGES = 800000
E_PAD = 819200
UPW = 128
MSGW = 144
SCW = 88
NR = 8
NRN = 6272
TRF = NRN // 16

_BN = 2000
_BE = 2560

_SQ2 = float(np.sqrt(2.0))
_SQ3 = float(np.sqrt(3.0))
_SQ5 = float(np.sqrt(5.0))
_SQ15 = float(np.sqrt(15.0))


def _swish(x):
    return x / (1.0 + jnp.exp(-x))


def _node_prep_body(x_ref, sp_ref, wup_ref, wsc_ref, up_ref, sc_ref):
    x = x_ref[...]
    sp = sp_ref[...]
    up_ref[...] = jnp.dot(x, wup_ref[...], preferred_element_type=jnp.float32)
    wsc = wsc_ref[...]
    acc = jnp.zeros((x.shape[0], SCW), jnp.float32)
    for k in range(5):
        xk = jnp.where(sp == k, x, 0.0)
        acc = acc + jnp.dot(xk, wsc[72 * k:72 * (k + 1), :],
                            preferred_element_type=jnp.float32)
    sc_ref[...] = acc


def _node_prep(node_feats, specie2, w_up_full, w_sc_full):
    n_blocks = N_NODES // _BN
    return pl.pallas_call(
        _node_prep_body,
        grid=(n_blocks,),
        in_specs=[
            pl.BlockSpec((_BN, 9 * FD), lambda i: (i, 0)),
            pl.BlockSpec((_BN, 1), lambda i: (i, 0)),
            pl.BlockSpec((9 * FD, UPW), lambda i: (0, 0)),
            pl.BlockSpec((360, SCW), lambda i: (0, 0)),
        ],
        out_specs=[
            pl.BlockSpec((_BN, UPW), lambda i: (i, 0)),
            pl.BlockSpec((_BN, SCW), lambda i: (i, 0)),
        ],
        out_shape=[
            jax.ShapeDtypeStruct((N_NODES, UPW), jnp.float32),
            jax.ShapeDtypeStruct((N_NODES, SCW), jnp.float32),
        ],
    )(node_feats, specie2, w_up_full, w_sc_full)


def _make_gather():
    mesh = plsc.VectorSubcoreMesh(core_axis_name="c", subcore_axis_name="s")
    n_chunks = E_PAD // (32 * 512)

    @functools.partial(
        pl.kernel,
        out_type=jax.ShapeDtypeStruct((E_PAD, UPW), jnp.float32),
        mesh=mesh,
        scratch_types=[
            pltpu.VMEM((4, 128), jnp.int32),
            pltpu.VMEM((512, UPW), jnp.float32),
            pltpu.SemaphoreType.DMA,
        ],
    )
    def gather_k(up_hbm, snd2_hbm, ef_hbm, idx2_v, rows_v, sem):
        wid = lax.axis_index("s") * 2 + lax.axis_index("c")

        def chunk(i, carry):
            rowbase = wid * (n_chunks * 4) + i * 4
            pltpu.sync_copy(snd2_hbm.at[pl.ds(rowbase, 4)], idx2_v)
            descs = [
                pltpu.async_copy(up_hbm.at[idx2_v.at[b]],
                                 rows_v.at[pl.ds(b * 128, 128)], sem)
                for b in range(4)
            ]
            for d in descs:
                d.wait()
            pltpu.sync_copy(
                rows_v, ef_hbm.at[pl.ds(wid * (n_chunks * 512) + i * 512, 512)])
            return carry

        lax.fori_loop(0, n_chunks, chunk, 0)

    return gather_k


def _edge_msg_body(vec_ref, ef_ref, w0_ref, b0_ref, w1_ref, b1_ref, w2_ref,
                   lo_ref, hi_ref):
    v3 = vec_ref[...]
    x_ = v3[:, 0:1]
    y_ = v3[:, 1:2]
    z_ = v3[:, 2:3]
    r = jnp.sqrt(x_ * x_ + y_ * y_ + z_ * z_)
    rinv = 1.0 / jnp.maximum(r, 1e-9)
    ux, uy, uz = x_ * rinv, y_ * rinv, z_ * rinv
    y1 = (_SQ3 * ux, _SQ3 * uy, _SQ3 * uz)
    y2 = (_SQ15 * ux * uy,
          _SQ15 * uy * uz,
          (_SQ5 / 2.0) * (3.0 * uz * uz - 1.0),
          _SQ15 * ux * uz,
          (_SQ15 / 2.0) * (ux * ux - uy * uy))

    xc = jnp.maximum(r, 1e-9)
    ns = lax.broadcasted_iota(jnp.int32, (1, FD), 1).astype(jnp.float32) + 1.0
    bes = _SQ2 * jnp.sin(jnp.pi * xc * ns) / xc
    p = 6.0
    a_c = -(p + 1.0) * (p + 2.0) / 2.0
    b_c = p * (p + 2.0)
    c_c = -p * (p + 1.0) / 2.0
    r2 = r * r
    r6 = r2 * r2 * r2
    env = jnp.where(r < 1.0, 1.0 + r6 * (a_c + r * b_c + r2 * c_c), 0.0)
    rad = bes * env

    h = _swish(jnp.dot(rad, w0_ref[...], preferred_element_type=jnp.float32)
               + b0_ref[...])
    h = _swish(jnp.dot(h, w1_ref[...], preferred_element_type=jnp.float32)
               + b1_ref[...])
    mix = jnp.dot(h, w2_ref[...], preferred_element_type=jnp.float32)

    ef = ef_ref[...]
    ms = ef[:, 0:FD]
    mv = [ef[:, FD + 8 * c:FD + 8 * (c + 1)] for c in range(3)]
    mt = [ef[:, 32 + 8 * c:32 + 8 * (c + 1)] for c in range(5)]
    mvu = (mv[0] * ux + mv[1] * uy + mv[2] * uz)

    pieces = [ms * mix[:, 0:8], mvu * mix[:, 8:16]]
    for c in range(3):
        pieces.append(mv[c] * mix[:, 16:24])
        pieces.append(ms * y1[c] * mix[:, 24:32])
    for c in range(5):
        pieces.append(mt[c] * mix[:, 32:40])
        pieces.append(ms * y2[c] * mix[:, 40:48])
    lo_ref[...] = jnp.concatenate(pieces[:16], axis=1)
    hi_ref[...] = jnp.concatenate(
        pieces[16:] + [jnp.zeros((v3.shape[0], 112), jnp.float32)], axis=1)


def _edge_msg(vec_p, ef, w0, b0, w1, b1, w2):
    n_blocks = E_PAD // _BE
    return pl.pallas_call(
        _edge_msg_body,
        grid=(n_blocks,),
        in_specs=[
            pl.BlockSpec((_BE, 3), lambda i: (i, 0)),
            pl.BlockSpec((_BE, UPW), lambda i: (i, 0)),
            pl.BlockSpec((FD, 64), lambda i: (0, 0)),
            pl.BlockSpec((1, 64), lambda i: (0, 0)),
            pl.BlockSpec((64, 64), lambda i: (0, 0)),
            pl.BlockSpec((1, 64), lambda i: (0, 0)),
            pl.BlockSpec((64, 48), lambda i: (0, 0)),
        ],
        out_specs=[pl.BlockSpec((_BE, 128), lambda i: (i, 0)),
                   pl.BlockSpec((_BE, 128), lambda i: (i, 0))],
        out_shape=[jax.ShapeDtypeStruct((E_PAD, 128), jnp.float32),
                   jax.ShapeDtypeStruct((E_PAD, 128), jnp.float32)],
    )(vec_p, ef, w0, b0, w1, b1, w2)


def _make_scatter():
    mesh = plsc.VectorSubcoreMesh(core_axis_name="c", subcore_axis_name="s")
    et = E_PAD // 16
    n_chunks = et // 512

    @functools.partial(
        pl.kernel,
        out_type=[jax.ShapeDtypeStruct((NR * NRN, 128), jnp.float32),
                  jax.ShapeDtypeStruct((NR * NRN, 128), jnp.float32)],
        mesh=mesh,
        scratch_types=[
            pltpu.VMEM((4, 128), jnp.int32),
            pltpu.VMEM((4, 128), jnp.int32),
            pltpu.VMEM((512, 128), jnp.float32),
            pltpu.VMEM_SHARED((NRN + 8, 128), jnp.float32),
        ],
    )
    def scatter_k(lo_hbm, hi_hbm, rcv2_hbm, z_hbm, agg_lo_hbm, agg_hi_hbm,
                  rcv_v, idx_v, msg_v, slab):
        cid = lax.axis_index("c")
        sid = lax.axis_index("s")

        for msg_hbm, agg_hbm in ((lo_hbm, agg_lo_hbm), (hi_hbm, agg_hi_hbm)):
            for p_ in range(NR // 2):
                lo = (cid * (NR // 2) + p_) * NRN
                hi = lo + NRN

                pltpu.sync_copy(z_hbm, slab.at[pl.ds(sid * TRF, TRF)])

                @pl.when(sid == 0)
                def _zero_trash():
                    pltpu.sync_copy(z_hbm.at[pl.ds(0, 8)], slab.at[pl.ds(NRN, 8)])

                plsc.subcore_barrier()

                def chunk(j, carry):
                    pltpu.sync_copy(
                        rcv2_hbm.at[pl.ds(sid * (n_chunks * 4) + j * 4, 4)], rcv_v)
                    pltpu.sync_copy(msg_hbm.at[pl.ds(sid * et + j * 512, 512)],
                                    msg_v)
                    for b in range(4):
                        for g in range(8):
                            rv = rcv_v[b, pl.ds(g * 16, 16)]
                            ok = (rv >= lo) & (rv < hi)
                            idx_v[b, pl.ds(g * 16, 16)] = jnp.where(ok, rv - lo,
                                                                    NRN)
                    for b in range(4):
                        pltpu.sync_copy(msg_v.at[pl.ds(b * 128, 128)],
                                        slab.at[idx_v.at[b]], add=True)
                    return carry

                lax.fori_loop(0, n_chunks, chunk, 0)
                plsc.subcore_barrier()
                pltpu.sync_copy(slab.at[pl.ds(sid * TRF, TRF)],
                                agg_hbm.at[pl.ds(lo + sid * TRF, TRF)])
                plsc.subcore_barrier()

    return scatter_k


def _node_post_body(agg_lo_ref, agg_hi_ref, sc_ref, wdn_ref, perm_ref, out_ref):
    a = jnp.concatenate([agg_lo_ref[...], agg_hi_ref[:, 0:16]], axis=1)
    sc = sc_ref[...]
    pre = jnp.dot(a, wdn_ref[...], preferred_element_type=jnp.float32) + sc
    feat_s = _swish(pre[:, 0:8])
    gates = _swish(pre[:, 8:24])
    gv = gates[:, 0:8]
    gt = gates[:, 8:16]
    pieces = [feat_s]
    for c in range(3):
        pieces.append(pre[:, 24 + 8 * c:32 + 8 * c] * gv)
    for c in range(5):
        pieces.append(pre[:, 48 + 8 * c:56 + 8 * c] * gt)
    cm = jnp.concatenate(pieces, axis=1)
    out_ref[...] = jnp.dot(cm, perm_ref[...], preferred_element_type=jnp.float32)


def _node_post(agg_lo, agg_hi, sc, w_dn_block, perm):
    n_blocks = N_NODES // _BN
    return pl.pallas_call(
        _node_post_body,
        grid=(n_blocks,),
        in_specs=[
            pl.BlockSpec((_BN, 128), lambda i: (i, 0)),
            pl.BlockSpec((_BN, 128), lambda i: (i, 0)),
            pl.BlockSpec((_BN, SCW), lambda i: (i, 0)),
            pl.BlockSpec((MSGW, SCW), lambda i: (0, 0)),
            pl.BlockSpec((72, 72), lambda i: (0, 0)),
        ],
        out_specs=pl.BlockSpec((_BN, 72), lambda i: (i, 0)),
        out_shape=jax.ShapeDtypeStruct((N_NODES, 72), jnp.float32),
    )(agg_lo, agg_hi, sc, w_dn_block, perm)


def _build_up_full(w_up_s, w_up_v, w_up_t):
    eye3 = jnp.eye(3, dtype=jnp.float32)
    eye5 = jnp.eye(5, dtype=jnp.float32)
    mv = (w_up_v[:, None, None, :] * eye3[None, :, :, None]).reshape(24, 24)
    mt = (w_up_t[:, None, None, :] * eye5[None, :, :, None]).reshape(40, 40)
    r_s = jnp.concatenate([w_up_s, jnp.zeros((8, UPW - 8), jnp.float32)], axis=1)
    r_v = jnp.concatenate([jnp.zeros((24, 8), jnp.float32), mv,
                           jnp.zeros((24, UPW - 32), jnp.float32)], axis=1)
    r_t = jnp.concatenate([jnp.zeros((40, 32), jnp.float32), mt,
                           jnp.zeros((40, UPW - 72), jnp.float32)], axis=1)
    return jnp.concatenate([r_s, r_v, r_t], axis=0)


def _build_sc_full(w_sc_s, w_sc_v, w_sc_t):
    eye3 = jnp.eye(3, dtype=jnp.float32)
    eye5 = jnp.eye(5, dtype=jnp.float32)
    mv = (w_sc_v[:, :, None, None, :]
          * eye3[None, None, :, :, None]).reshape(5, 24, 24)
    mt = (w_sc_t[:, :, None, None, :]
          * eye5[None, None, :, :, None]).reshape(5, 40, 40)
    r_s = jnp.concatenate([w_sc_s, jnp.zeros((5, 8, 64), jnp.float32)], axis=2)
    r_v = jnp.concatenate([jnp.zeros((5, 24, 24), jnp.float32), mv,
                           jnp.zeros((5, 24, 40), jnp.float32)], axis=2)
    r_t = jnp.concatenate([jnp.zeros((5, 40, 48), jnp.float32), mt], axis=2)
    return jnp.concatenate([r_s, r_v, r_t], axis=1).reshape(360, SCW)


def _build_dn_block(w_dn_s, w_dn_v, w_dn_t):
    eye3 = jnp.eye(3, dtype=jnp.float32)
    eye5 = jnp.eye(5, dtype=jnp.float32)
    mv = (eye3[:, None, :, None] * w_dn_v[None, :, None, :]).reshape(48, 24)
    mt = (eye5[:, None, :, None] * w_dn_t[None, :, None, :]).reshape(80, 40)
    r_s = jnp.concatenate([w_dn_s, jnp.zeros((16, 64), jnp.float32)], axis=1)
    r_v = jnp.concatenate([jnp.zeros((48, 24), jnp.float32), mv,
                           jnp.zeros((48, 40), jnp.float32)], axis=1)
    r_t = jnp.concatenate([jnp.zeros((80, 48), jnp.float32), mt], axis=1)
    inv = 1.0 / 4.0
    return inv * jnp.concatenate([r_s, r_v, r_t], axis=0)


def _build_perm():
    p = np.zeros((72, 72), np.float32)
    for i in range(8):
        p[i, i] = 1.0
    for c in range(3):
        for o in range(8):
            p[8 + c * 8 + o, 8 + o * 3 + c] = 1.0
    for c in range(5):
        for o in range(8):
            p[32 + c * 8 + o, 32 + o * 5 + c] = 1.0
    return jnp.asarray(p)


def kernel(vectors, node_feats, node_specie, senders, receivers,
           W_up_s, W_up_v, W_up_t, mlp_w0, mlp_b0, mlp_w1, mlp_b1, mlp_w2,
           W_dn_s, W_dn_v, W_dn_t, W_sc_s, W_sc_v, W_sc_t):
    pad_e = E_PAD - E_EDGES
    snd2 = jnp.pad(senders.astype(jnp.int32), (0, pad_e)).reshape(-1, 128)
    rcv2 = jnp.pad(receivers.astype(jnp.int32), (0, pad_e),
                   constant_values=1 << 20).reshape(-1, 128)
    vec_p = jnp.pad(vectors, ((0, pad_e), (0, 0)))
    specie2 = node_specie.astype(jnp.int32).reshape(N_NODES, 1)

    w_up_full = _build_up_full(W_up_s, W_up_v, W_up_t)
    w_sc_full = _build_sc_full(W_sc_s, W_sc_v, W_sc_t)
    w_dn_block = _build_dn_block(W_dn_s, W_dn_v, W_dn_t)
    perm = _build_perm()
    z_hbm = jnp.zeros((TRF, 128), jnp.float32)

    up, sc = _node_prep(node_feats, specie2, w_up_full, w_sc_full)
    ef = _make_gather()(up, snd2)
    msg_lo, msg_hi = _edge_msg(vec_p, ef, mlp_w0, mlp_b0.reshape(1, 64),
                               mlp_w1, mlp_b1.reshape(1, 64), mlp_w2)
    agg_lo, agg_hi = _make_scatter()(msg_lo, msg_hi, rcv2, z_hbm)
    return _node_post(agg_lo, agg_hi, sc, w_dn_block, perm)


# speedup vs baseline: 11.8798x; 1.1659x over previous
import functools

import jax
import jax.numpy as jnp
import numpy as np
from jax import lax
from jax.experimental import pallas as pl
from jax.experimental.pallas import tpu as pltpu
from jax.experimental.pallas import tpu_sc as plsc

FD = 8
N_NODES = 50000
E_EDGES = 800000
E_PAD = 819200
UPW = 128
MSGW = 144
SCW = 88
NR = 8
NRN = 6272
TRF = NRN // 16

_BN = 2000
_BE = 2560

_SQ2 = float(np.sqrt(2.0))
_SQ3 = float(np.sqrt(3.0))
_SQ5 = float(np.sqrt(5.0))
_SQ15 = float(np.sqrt(15.0))


def _swish(x):
    return x / (1.0 + jnp.exp(-x))


def _node_prep_body(x_ref, sp_ref, wup_ref, wsc_ref, up_ref, sc_ref):
    x = x_ref[...]
    sp = sp_ref[...]
    up_ref[...] = jnp.dot(x, wup_ref[...], preferred_element_type=jnp.float32)
    wsc = wsc_ref[...]
    acc = jnp.zeros((x.shape[0], SCW), jnp.float32)
    for k in range(5):
        xk = jnp.where(sp == k, x, 0.0)
        acc = acc + jnp.dot(xk, wsc[72 * k:72 * (k + 1), :],
                            preferred_element_type=jnp.float32)
    sc_ref[...] = acc


def _node_prep(node_feats, specie2, w_up_full, w_sc_full):
    n_blocks = N_NODES // _BN
    return pl.pallas_call(
        _node_prep_body,
        grid=(n_blocks,),
        in_specs=[
            pl.BlockSpec((_BN, 9 * FD), lambda i: (i, 0)),
            pl.BlockSpec((_BN, 1), lambda i: (i, 0)),
            pl.BlockSpec((9 * FD, UPW), lambda i: (0, 0)),
            pl.BlockSpec((360, SCW), lambda i: (0, 0)),
        ],
        out_specs=[
            pl.BlockSpec((_BN, UPW), lambda i: (i, 0)),
            pl.BlockSpec((_BN, SCW), lambda i: (i, 0)),
        ],
        out_shape=[
            jax.ShapeDtypeStruct((N_NODES, UPW), jnp.float32),
            jax.ShapeDtypeStruct((N_NODES, SCW), jnp.float32),
        ],
    )(node_feats, specie2, w_up_full, w_sc_full)


def _make_gather():
    mesh = plsc.VectorSubcoreMesh(core_axis_name="c", subcore_axis_name="s")
    ch = 256
    n_chunks = E_PAD // (32 * ch)

    @functools.partial(
        pl.kernel,
        out_type=jax.ShapeDtypeStruct((E_PAD, UPW), jnp.float32),
        mesh=mesh,
        scratch_types=[
            pltpu.VMEM((2, 128), jnp.int32),
            pltpu.VMEM((2, 128), jnp.int32),
            pltpu.VMEM((ch, UPW), jnp.float32),
            pltpu.VMEM((ch, UPW), jnp.float32),
            pltpu.SemaphoreType.DMA,
            pltpu.SemaphoreType.DMA,
        ],
    )
    def gather_k(up_hbm, snd2_hbm, ef_hbm, idx0, idx1, rows0, rows1, sem0, sem1):
        wid = lax.axis_index("s") * 2 + lax.axis_index("c")
        idxs = (idx0, idx1)
        rows = (rows0, rows1)
        sems = (sem0, sem1)

        def fire(j, par):
            pltpu.sync_copy(snd2_hbm.at[pl.ds(wid * (n_chunks * 2) + j * 2, 2)],
                            idxs[par])
            for b in range(2):
                pltpu.async_copy(up_hbm.at[idxs[par].at[b]],
                                 rows[par].at[pl.ds(b * 128, 128)], sems[par])

        def drain(par):
            for b in range(2):
                pltpu.make_async_copy(up_hbm.at[idxs[par].at[b]],
                                      rows[par].at[pl.ds(b * 128, 128)],
                                      sems[par]).wait()

        def flush(j, par):
            pltpu.sync_copy(rows[par],
                            ef_hbm.at[pl.ds(wid * (n_chunks * ch) + j * ch, ch)])

        fire(0, 0)

        def step(i, carry):
            j0 = 2 * i
            drain(0)
            fire(j0 + 1, 1)
            flush(j0, 0)
            drain(1)

            @pl.when(i < n_chunks // 2 - 1)
            def _refire():
                fire(j0 + 2, 0)

            flush(j0 + 1, 1)
            return carry

        lax.fori_loop(0, n_chunks // 2, step, 0)

    return gather_k


def _edge_msg_body(vec_ref, eye_ref, ef_ref, w0_ref, b0_ref, w1_ref, b1_ref,
                   w2_ref, lo_ref, hi_ref):
    v3 = lax.dot_general(vec_ref[...], eye_ref[...], (((0,), (0,)), ((), ())),
                         precision=lax.Precision.HIGHEST,
                         preferred_element_type=jnp.float32)
    x_ = v3[:, 0:1]
    y_ = v3[:, 1:2]
    z_ = v3[:, 2:3]
    r = jnp.sqrt(x_ * x_ + y_ * y_ + z_ * z_)
    rinv = 1.0 / jnp.maximum(r, 1e-9)
    ux, uy, uz = x_ * rinv, y_ * rinv, z_ * rinv
    y1 = (_SQ3 * ux, _SQ3 * uy, _SQ3 * uz)
    y2 = (_SQ15 * ux * uy,
          _SQ15 * uy * uz,
          (_SQ5 / 2.0) * (3.0 * uz * uz - 1.0),
          _SQ15 * ux * uz,
          (_SQ15 / 2.0) * (ux * ux - uy * uy))

    xc = jnp.maximum(r, 1e-9)
    ns = lax.broadcasted_iota(jnp.int32, (1, FD), 1).astype(jnp.float32) + 1.0
    bes = _SQ2 * jnp.sin(jnp.pi * xc * ns) / xc
    p = 6.0
    a_c = -(p + 1.0) * (p + 2.0) / 2.0
    b_c = p * (p + 2.0)
    c_c = -p * (p + 1.0) / 2.0
    r2 = r * r
    r6 = r2 * r2 * r2
    env = jnp.where(r < 1.0, 1.0 + r6 * (a_c + r * b_c + r2 * c_c), 0.0)
    rad = bes * env

    h = _swish(jnp.dot(rad, w0_ref[...], preferred_element_type=jnp.float32)
               + b0_ref[...])
    h = _swish(jnp.dot(h, w1_ref[...], preferred_element_type=jnp.float32)
               + b1_ref[...])
    mix = jnp.dot(h, w2_ref[...], preferred_element_type=jnp.float32)

    ef = ef_ref[...]
    ms = ef[:, 0:FD]
    mv = [ef[:, FD + 8 * c:FD + 8 * (c + 1)] for c in range(3)]
    mt = [ef[:, 32 + 8 * c:32 + 8 * (c + 1)] for c in range(5)]
    mvu = (mv[0] * ux + mv[1] * uy + mv[2] * uz)

    pieces = [ms * mix[:, 0:8], mvu * mix[:, 8:16]]
    for c in range(3):
        pieces.append(mv[c] * mix[:, 16:24])
        pieces.append(ms * y1[c] * mix[:, 24:32])
    for c in range(5):
        pieces.append(mt[c] * mix[:, 32:40])
        pieces.append(ms * y2[c] * mix[:, 40:48])
    lo_ref[...] = jnp.concatenate(pieces[:16], axis=1)
    hi_ref[...] = jnp.concatenate(
        pieces[16:] + [jnp.zeros((v3.shape[0], 112), jnp.float32)], axis=1)


def _edge_msg(vec_p, ef, w0, b0, w1, b1, w2):
    n_blocks = E_PAD // _BE
    return pl.pallas_call(
        _edge_msg_body,
        grid=(n_blocks,),
        in_specs=[
            pl.BlockSpec((8, _BE), lambda i: (0, i)),
            pl.BlockSpec((8, 8), lambda i: (0, 0)),
            pl.BlockSpec((_BE, UPW), lambda i: (i, 0)),
            pl.BlockSpec((FD, 64), lambda i: (0, 0)),
            pl.BlockSpec((1, 64), lambda i: (0, 0)),
            pl.BlockSpec((64, 64), lambda i: (0, 0)),
            pl.BlockSpec((1, 64), lambda i: (0, 0)),
            pl.BlockSpec((64, 48), lambda i: (0, 0)),
        ],
        out_specs=[pl.BlockSpec((_BE, 128), lambda i: (i, 0)),
                   pl.BlockSpec((_BE, 128), lambda i: (i, 0))],
        out_shape=[jax.ShapeDtypeStruct((E_PAD, 128), jnp.float32),
                   jax.ShapeDtypeStruct((E_PAD, 128), jnp.float32)],
    )(vec_p, jnp.eye(8, dtype=jnp.float32), ef, w0, b0, w1, b1, w2)


def _make_scatter():
    mesh = plsc.VectorSubcoreMesh(core_axis_name="c", subcore_axis_name="s")
    ch = 256
    et = E_PAD // 16
    n_chunks = et // ch

    @functools.partial(
        pl.kernel,
        out_type=[jax.ShapeDtypeStruct((NR * NRN, 128), jnp.float32),
                  jax.ShapeDtypeStruct((NR * NRN, 128), jnp.float32)],
        mesh=mesh,
        scratch_types=[
            pltpu.VMEM((2, 128), jnp.int32),
            pltpu.VMEM((2, 128), jnp.int32),
            pltpu.VMEM((2, 128), jnp.int32),
            pltpu.VMEM((2, 128), jnp.int32),
            pltpu.VMEM((ch, 128), jnp.float32),
            pltpu.VMEM((ch, 128), jnp.float32),
            pltpu.VMEM_SHARED((NRN + 8, 128), jnp.float32),
            pltpu.SemaphoreType.DMA,
            pltpu.SemaphoreType.DMA,
        ],
    )
    def scatter_k(lo_hbm, hi_hbm, rcv2_hbm, z_hbm, agg_lo_hbm, agg_hi_hbm,
                  rcv0, rcv1, idx0, idx1, msg0, msg1, slab, sem0, sem1):
        cid = lax.axis_index("c")
        sid = lax.axis_index("s")
        rcvs = (rcv0, rcv1)
        idxs = (idx0, idx1)
        msgs = (msg0, msg1)
        sems = (sem0, sem1)

        for msg_hbm, agg_hbm in ((lo_hbm, agg_lo_hbm), (hi_hbm, agg_hi_hbm)):
            for p_ in range(NR // 2):
                lo = (cid * (NR // 2) + p_) * NRN
                hi = lo + NRN

                pltpu.sync_copy(z_hbm, slab.at[pl.ds(sid * TRF, TRF)])

                @pl.when(sid == 0)
                def _zero_trash():
                    pltpu.sync_copy(z_hbm.at[pl.ds(0, 8)], slab.at[pl.ds(NRN, 8)])

                plsc.subcore_barrier()

                def fire(j, par):
                    pltpu.async_copy(
                        rcv2_hbm.at[pl.ds(sid * (n_chunks * 2) + j * 2, 2)],
                        rcvs[par], sems[par])
                    pltpu.async_copy(msg_hbm.at[pl.ds(sid * et + j * ch, ch)],
                                     msgs[par], sems[par])

                def drain(par):
                    pltpu.make_async_copy(rcv2_hbm.at[pl.ds(0, 2)], rcvs[par],
                                          sems[par]).wait()
                    pltpu.make_async_copy(msg_hbm.at[pl.ds(0, ch)], msgs[par],
                                          sems[par]).wait()

                def process(par):
                    for b in range(2):
                        for g in range(8):
                            rv = rcvs[par][b, pl.ds(g * 16, 16)]
                            ok = (rv >= lo) & (rv < hi)
                            idxs[par][b, pl.ds(g * 16, 16)] = jnp.where(
                                ok, rv - lo, NRN)
                    for b in range(2):
                        pltpu.sync_copy(msgs[par].at[pl.ds(b * 128, 128)],
                                        slab.at[idxs[par].at[b]], add=True)

                fire(0, 0)

                def step(i, carry):
                    j0 = 2 * i
                    drain(0)
                    fire(j0 + 1, 1)
                    process(0)
                    drain(1)

                    @pl.when(i < n_chunks // 2 - 1)
                    def _refire():
                        fire(j0 + 2, 0)

                    process(1)
                    return carry

                lax.fori_loop(0, n_chunks // 2, step, 0)
                plsc.subcore_barrier()
                pltpu.sync_copy(slab.at[pl.ds(sid * TRF, TRF)],
                                agg_hbm.at[pl.ds(lo + sid * TRF, TRF)])
                plsc.subcore_barrier()

    return scatter_k


def _node_post_body(agg_lo_ref, agg_hi_ref, sc_ref, wdn_ref, perm_ref, out_ref):
    a = jnp.concatenate([agg_lo_ref[...], agg_hi_ref[:, 0:16]], axis=1)
    sc = sc_ref[...]
    pre = jnp.dot(a, wdn_ref[...], preferred_element_type=jnp.float32) + sc
    feat_s = _swish(pre[:, 0:8])
    gates = _swish(pre[:, 8:24])
    gv = gates[:, 0:8]
    gt = gates[:, 8:16]
    pieces = [feat_s]
    for c in range(3):
        pieces.append(pre[:, 24 + 8 * c:32 + 8 * c] * gv)
    for c in range(5):
        pieces.append(pre[:, 48 + 8 * c:56 + 8 * c] * gt)
    cm = jnp.concatenate(pieces, axis=1)
    out_ref[...] = jnp.dot(cm, perm_ref[...], preferred_element_type=jnp.float32)


def _node_post(agg_lo, agg_hi, sc, w_dn_block, perm):
    n_blocks = N_NODES // _BN
    return pl.pallas_call(
        _node_post_body,
        grid=(n_blocks,),
        in_specs=[
            pl.BlockSpec((_BN, 128), lambda i: (i, 0)),
            pl.BlockSpec((_BN, 128), lambda i: (i, 0)),
            pl.BlockSpec((_BN, SCW), lambda i: (i, 0)),
            pl.BlockSpec((MSGW, SCW), lambda i: (0, 0)),
            pl.BlockSpec((72, 72), lambda i: (0, 0)),
        ],
        out_specs=pl.BlockSpec((_BN, 72), lambda i: (i, 0)),
        out_shape=jax.ShapeDtypeStruct((N_NODES, 72), jnp.float32),
    )(agg_lo, agg_hi, sc, w_dn_block, perm)


def _build_up_full(w_up_s, w_up_v, w_up_t):
    eye3 = jnp.eye(3, dtype=jnp.float32)
    eye5 = jnp.eye(5, dtype=jnp.float32)
    mv = (w_up_v[:, None, None, :] * eye3[None, :, :, None]).reshape(24, 24)
    mt = (w_up_t[:, None, None, :] * eye5[None, :, :, None]).reshape(40, 40)
    r_s = jnp.concatenate([w_up_s, jnp.zeros((8, UPW - 8), jnp.float32)], axis=1)
    r_v = jnp.concatenate([jnp.zeros((24, 8), jnp.float32), mv,
                           jnp.zeros((24, UPW - 32), jnp.float32)], axis=1)
    r_t = jnp.concatenate([jnp.zeros((40, 32), jnp.float32), mt,
                           jnp.zeros((40, UPW - 72), jnp.float32)], axis=1)
    return jnp.concatenate([r_s, r_v, r_t], axis=0)


def _build_sc_full(w_sc_s, w_sc_v, w_sc_t):
    eye3 = jnp.eye(3, dtype=jnp.float32)
    eye5 = jnp.eye(5, dtype=jnp.float32)
    mv = (w_sc_v[:, :, None, None, :]
          * eye3[None, None, :, :, None]).reshape(5, 24, 24)
    mt = (w_sc_t[:, :, None, None, :]
          * eye5[None, None, :, :, None]).reshape(5, 40, 40)
    r_s = jnp.concatenate([w_sc_s, jnp.zeros((5, 8, 64), jnp.float32)], axis=2)
    r_v = jnp.concatenate([jnp.zeros((5, 24, 24), jnp.float32), mv,
                           jnp.zeros((5, 24, 40), jnp.float32)], axis=2)
    r_t = jnp.concatenate([jnp.zeros((5, 40, 48), jnp.float32), mt], axis=2)
    return jnp.concatenate([r_s, r_v, r_t], axis=1).reshape(360, SCW)


def _build_dn_block(w_dn_s, w_dn_v, w_dn_t):
    eye3 = jnp.eye(3, dtype=jnp.float32)
    eye5 = jnp.eye(5, dtype=jnp.float32)
    mv = (eye3[:, None, :, None] * w_dn_v[None, :, None, :]).reshape(48, 24)
    mt = (eye5[:, None, :, None] * w_dn_t[None, :, None, :]).reshape(80, 40)
    r_s = jnp.concatenate([w_dn_s, jnp.zeros((16, 64), jnp.float32)], axis=1)
    r_v = jnp.concatenate([jnp.zeros((48, 24), jnp.float32), mv,
                           jnp.zeros((48, 40), jnp.float32)], axis=1)
    r_t = jnp.concatenate([jnp.zeros((80, 48), jnp.float32), mt], axis=1)
    inv = 1.0 / 4.0
    return inv * jnp.concatenate([r_s, r_v, r_t], axis=0)


def _build_perm():
    p = np.zeros((72, 72), np.float32)
    for i in range(8):
        p[i, i] = 1.0
    for c in range(3):
        for o in range(8):
            p[8 + c * 8 + o, 8 + o * 3 + c] = 1.0
    for c in range(5):
        for o in range(8):
            p[32 + c * 8 + o, 32 + o * 5 + c] = 1.0
    return jnp.asarray(p)


def kernel(vectors, node_feats, node_specie, senders, receivers,
           W_up_s, W_up_v, W_up_t, mlp_w0, mlp_b0, mlp_w1, mlp_b1, mlp_w2,
           W_dn_s, W_dn_v, W_dn_t, W_sc_s, W_sc_v, W_sc_t):
    pad_e = E_PAD - E_EDGES
    snd2 = jnp.pad(senders.astype(jnp.int32), (0, pad_e)).reshape(-1, 128)
    rcv2 = jnp.pad(receivers.astype(jnp.int32), (0, pad_e),
                   constant_values=1 << 20).reshape(-1, 128)
    vec_p = jnp.pad(vectors.T, ((0, 5), (0, pad_e)))
    specie2 = node_specie.astype(jnp.int32).reshape(N_NODES, 1)

    w_up_full = _build_up_full(W_up_s, W_up_v, W_up_t)
    w_sc_full = _build_sc_full(W_sc_s, W_sc_v, W_sc_t)
    w_dn_block = _build_dn_block(W_dn_s, W_dn_v, W_dn_t)
    perm = _build_perm()
    z_hbm = jnp.zeros((TRF, 128), jnp.float32)

    up, sc = _node_prep(node_feats, specie2, w_up_full, w_sc_full)
    ef = _make_gather()(up, snd2)
    msg_lo, msg_hi = _edge_msg(vec_p, ef, mlp_w0, mlp_b0.reshape(1, 64),
                               mlp_w1, mlp_b1.reshape(1, 64), mlp_w2)
    agg_lo, agg_hi = _make_scatter()(msg_lo, msg_hi, rcv2, z_hbm)
    return _node_post(agg_lo, agg_hi, sc, w_dn_block, perm)
